# baseline (speedup 1.0000x reference)
import functools

import jax
import jax.numpy as jnp
from jax import lax
from jax.experimental import pallas as pl
from jax.experimental.pallas import tpu as pltpu
from jax.experimental.pallas import tpu_sc as plsc

VOCAB = 100000
EMB = 256
TOTAL = 16384

_NC = 2
_NS = 16
_NW = _NC * _NS
_B_PER_W = TOTAL // _NW
_CHUNK = 64
_N_CHUNKS = _B_PER_W // _CHUNK
_NBUF = 7

_mesh = plsc.VectorSubcoreMesh(core_axis_name="c", subcore_axis_name="s")


@functools.partial(
    pl.kernel,
    mesh=_mesh,
    out_type=jax.ShapeDtypeStruct((TOTAL, EMB), jnp.float32),
    scratch_types=[
        pltpu.VMEM((_B_PER_W,), jnp.int32),
    ]
    + [pltpu.VMEM((_CHUNK, EMB), jnp.float32) for _ in range(_NBUF)]
    + [pltpu.SemaphoreType.DMA for _ in range(2 * _NBUF)],
)
def _gather_kernel(tokens_hbm, table_hbm, out_hbm, idx_v, *bufs_sems):
    bufs = bufs_sems[:_NBUF]
    gsems = bufs_sems[_NBUF : 2 * _NBUF]
    wsems = bufs_sems[2 * _NBUF :]
    wid = lax.axis_index("s") * _NC + lax.axis_index("c")
    base = wid * _B_PER_W
    pltpu.sync_copy(tokens_hbm.at[pl.ds(base, _B_PER_W)], idx_v)

    gcp = [None] * _NBUF
    wcp = [None] * _NBUF
    for j in range(_N_CHUNKS):
        b = j % _NBUF
        if j >= _NBUF:
            wcp[b].wait()
        gcp[b] = pltpu.async_copy(
            table_hbm.at[idx_v.at[pl.ds(j * _CHUNK, _CHUNK)]], bufs[b], gsems[b]
        )
        d = j - (_NBUF - 1)
        if d >= 0:
            db = d % _NBUF
            gcp[db].wait()
            wcp[db] = pltpu.async_copy(
                bufs[db], out_hbm.at[pl.ds(base + d * _CHUNK, _CHUNK)], wsems[db]
            )
    for d in range(max(0, _N_CHUNKS - (_NBUF - 1)), _N_CHUNKS):
        db = d % _NBUF
        gcp[db].wait()
        wcp[db] = pltpu.async_copy(
            bufs[db], out_hbm.at[pl.ds(base + d * _CHUNK, _CHUNK)], wsems[db]
        )
    for d in range(max(0, _N_CHUNKS - _NBUF), _N_CHUNKS):
        wcp[d % _NBUF].wait()


def kernel(flat_tokens, cu_seqlens, table):
    del cu_seqlens
    return _gather_kernel(flat_tokens, table)


# speedup vs baseline: 1.0008x; 1.0008x over previous
import functools

import jax
import jax.numpy as jnp
from jax import lax
from jax.experimental import pallas as pl
from jax.experimental.pallas import tpu as pltpu
from jax.experimental.pallas import tpu_sc as plsc

VOCAB = 100000
EMB = 256
TOTAL = 16384

_NC = 2
_NS = 16
_NW = _NC * _NS
_B_PER_W = TOTAL // _NW
_CHUNK = 64
_N_CHUNKS = _B_PER_W // _CHUNK
_NBUF = 6

_mesh = plsc.VectorSubcoreMesh(core_axis_name="c", subcore_axis_name="s")


@functools.partial(
    pl.kernel,
    mesh=_mesh,
    out_type=jax.ShapeDtypeStruct((TOTAL, EMB), jnp.float32),
    scratch_types=[
        pltpu.VMEM((_B_PER_W,), jnp.int32),
    ]
    + [pltpu.VMEM((_CHUNK, EMB), jnp.float32) for _ in range(_NBUF)]
    + [pltpu.SemaphoreType.DMA for _ in range(2 * _NBUF + 1)],
)
def _gather_kernel(tokens_hbm, table_hbm, out_hbm, idx_v, *bufs_sems):
    bufs = bufs_sems[:_NBUF]
    gsems = bufs_sems[_NBUF : 2 * _NBUF]
    wsems = bufs_sems[2 * _NBUF : 3 * _NBUF]
    isem = bufs_sems[3 * _NBUF]
    wid = lax.axis_index("s") * _NC + lax.axis_index("c")
    base = wid * _B_PER_W
    pltpu.sync_copy(tokens_hbm.at[pl.ds(base, _CHUNK)], idx_v.at[pl.ds(0, _CHUNK)])
    icp = pltpu.async_copy(
        tokens_hbm.at[pl.ds(base + _CHUNK, _B_PER_W - _CHUNK)],
        idx_v.at[pl.ds(_CHUNK, _B_PER_W - _CHUNK)],
        isem,
    )

    gcp = [None] * _NBUF
    wcp = [None] * _NBUF
    for j in range(_N_CHUNKS):
        b = j % _NBUF
        if j >= _NBUF:
            wcp[b].wait()
        if j == 1:
            icp.wait()
        gcp[b] = pltpu.async_copy(
            table_hbm.at[idx_v.at[pl.ds(j * _CHUNK, _CHUNK)]], bufs[b], gsems[b]
        )
        d = j - (_NBUF - 1)
        if d >= 0:
            db = d % _NBUF
            gcp[db].wait()
            wcp[db] = pltpu.async_copy(
                bufs[db], out_hbm.at[pl.ds(base + d * _CHUNK, _CHUNK)], wsems[db]
            )
    for d in range(max(0, _N_CHUNKS - (_NBUF - 1)), _N_CHUNKS):
        db = d % _NBUF
        gcp[db].wait()
        wcp[db] = pltpu.async_copy(
            bufs[db], out_hbm.at[pl.ds(base + d * _CHUNK, _CHUNK)], wsems[db]
        )
    for d in range(max(0, _N_CHUNKS - _NBUF), _N_CHUNKS):
        wcp[d % _NBUF].wait()


def kernel(flat_tokens, cu_seqlens, table):
    del cu_seqlens
    return _gather_kernel(flat_tokens, table)


# speedup vs baseline: 1.0101x; 1.0093x over previous
import functools

import jax
import jax.numpy as jnp
from jax import lax
from jax.experimental import pallas as pl
from jax.experimental.pallas import tpu as pltpu
from jax.experimental.pallas import tpu_sc as plsc

VOCAB = 100000
EMB = 256
TOTAL = 16384

_NC = 2
_NS = 16
_NW = _NC * _NS
_B_PER_W = TOTAL // _NW
_CHUNK = 64
_N_CHUNKS = _B_PER_W // _CHUNK
_NBUF = 6

_mesh = plsc.VectorSubcoreMesh(core_axis_name="c", subcore_axis_name="s")


@functools.partial(
    pl.kernel,
    mesh=_mesh,
    out_type=jax.ShapeDtypeStruct((TOTAL, EMB), jnp.float32),
    scratch_types=[
        pltpu.VMEM((_B_PER_W,), jnp.int32),
    ]
    + [pltpu.VMEM((_CHUNK, EMB), jnp.float32) for _ in range(_NBUF)]
    + [pltpu.SemaphoreType.DMA for _ in range(2 * _NBUF)],
)
def _gather_kernel(tokens_hbm, table_hbm, out_hbm, idx_v, *bufs_sems):
    bufs = bufs_sems[:_NBUF]
    gsems = bufs_sems[_NBUF : 2 * _NBUF]
    wsems = bufs_sems[2 * _NBUF :]
    wid = lax.axis_index("s") * _NC + lax.axis_index("c")
    base = wid * _B_PER_W
    pltpu.sync_copy(tokens_hbm.at[pl.ds(base, _B_PER_W)], idx_v)

    gcp = [None] * _NBUF
    wcp = [None] * _NBUF
    for j in range(_N_CHUNKS):
        b = j % _NBUF
        if j >= _NBUF:
            wcp[b].wait()
        gcp[b] = pltpu.async_copy(
            table_hbm.at[idx_v.at[pl.ds(j * _CHUNK, _CHUNK)]], bufs[b], gsems[b]
        )
        d = j - (_NBUF - 1)
        if d >= 0:
            db = d % _NBUF
            gcp[db].wait()
            wcp[db] = pltpu.async_copy(
                bufs[db], out_hbm.at[pl.ds(base + d * _CHUNK, _CHUNK)], wsems[db]
            )
    for d in range(max(0, _N_CHUNKS - (_NBUF - 1)), _N_CHUNKS):
        db = d % _NBUF
        gcp[db].wait()
        wcp[db] = pltpu.async_copy(
            bufs[db], out_hbm.at[pl.ds(base + d * _CHUNK, _CHUNK)], wsems[db]
        )
    for d in range(max(0, _N_CHUNKS - _NBUF), _N_CHUNKS):
        wcp[d % _NBUF].wait()


def kernel(flat_tokens, cu_seqlens, table):
    del cu_seqlens
    return _gather_kernel(flat_tokens, table)
